# baseline (speedup 1.0000x reference)
import jax
import jax.numpy as jnp
from jax.experimental import pallas as pl

N = 10242
KS = 20
CIN = 16
COUT = 16
N_PAD = 10368


def _matmul_body(x_ref, m_ref, o_ref):
    k = pl.program_id(1)

    @pl.when(k == 0)
    def _():
        o_ref[...] = jnp.zeros_like(o_ref)

    o_ref[...] += jnp.dot(x_ref[0], m_ref[0],
                          preferred_element_type=jnp.float32)


def _final_matmul(xs, m):
    bn = 3456
    grid = (N_PAD // bn, KS)
    return pl.pallas_call(
        _matmul_body,
        grid=grid,
        in_specs=[
            pl.BlockSpec((1, bn, 256), lambda n, k: (k, n, 0)),
            pl.BlockSpec((1, 256, 256), lambda n, k: (k, 0, 0)),
        ],
        out_specs=pl.BlockSpec((bn, 256), lambda n, k: (n, 0)),
        out_shape=jax.ShapeDtypeStruct((N_PAD, 256), jnp.float32),
    )(xs, m)


def kernel(tensor, edge_row, edge_col, edge_val, W):
    Bv, Tv, Nv, Cv = tensor.shape
    BT = Bv * Tv
    x = jnp.reshape(tensor, (BT, Nv, Cv))
    x = jnp.transpose(x, (1, 2, 0))
    x0 = jnp.reshape(x, (Nv, Cv * BT))

    def spmv(v):
        return jax.ops.segment_sum(edge_val[:, None] * v[edge_col], edge_row,
                                   num_segments=Nv)

    xs = [x0]
    x1 = spmv(x0)
    xs.append(x1)
    xp0, xp1 = x0, x1
    for _ in range(2, KS):
        x2 = 2.0 * spmv(xp1) - xp0
        xs.append(x2)
        xp0, xp1 = xp1, x2
    xk = jnp.stack(xs, axis=0)
    xk = jnp.pad(xk, ((0, 0), (0, N_PAD - Nv), (0, 0)))

    w3 = W.reshape(COUT, CIN, KS)
    m = jnp.einsum('ock,ij->kcioj', w3, jnp.eye(BT, dtype=W.dtype))
    m = m.reshape(KS, Cv * BT, COUT * BT)

    out = _final_matmul(xk, m)[:Nv]
    out = out.reshape(Nv, COUT, BT)
    out = jnp.transpose(out, (2, 0, 1))
    return out.reshape(Bv, Tv, Nv, COUT)


# speedup vs baseline: 2.0732x; 2.0732x over previous
import dataclasses
import functools

import jax
import jax.numpy as jnp
from jax import lax
from jax.experimental import pallas as pl
from jax.experimental.pallas import tpu as pltpu
from jax.experimental.pallas import tpu_sc as plsc

N = 10242
E = 71694
KS = 20
CIN = 16
COUT = 16
F = 256
N_PAD = 10368
R = 324
NW = 32
CH = 64
E_PAD = ((E + CH - 1) // CH) * CH

_mesh = plsc.VectorSubcoreMesh(core_axis_name="c", subcore_axis_name="s")

_sc_params = pltpu.CompilerParams()
if "needs_layout_passes" in pltpu.CompilerParams.__dataclass_fields__:
    _sc_params = dataclasses.replace(_sc_params, needs_layout_passes=False)


def _make_sc_step(first):
    @functools.partial(
        pl.kernel,
        mesh=_mesh,
        out_type=jax.ShapeDtypeStruct((N_PAD * F,), jnp.float32),
        scratch_types=[
            pltpu.VMEM((R * F,), jnp.float32),
            pltpu.VMEM((CH, F), jnp.float32),
            pltpu.VMEM((CH,), jnp.int32),
            pltpu.VMEM((CH,), jnp.int32),
            pltpu.VMEM((CH,), jnp.float32),
            pltpu.VMEM((CH,), jnp.int32),
            pltpu.VMEM((CH,), jnp.float32),
            pltpu.VMEM((16,), jnp.int32),
            pltpu.SemaphoreType.DMA,
        ],
        compiler_params=_sc_params,
    )
    def step(xg_hbm, xi_hbm, cols_hbm, rows_hbm, vals_hbm, bounds_hbm,
             xo_hbm, acc, xbuf, colb, rowb, valb, lrowb, veffb, bndb, sem):
        cid = lax.axis_index("c")
        sid = lax.axis_index("s")
        wid = sid * 2 + cid
        base = wid * R
        off = base * F
        iota = lax.iota(jnp.int32, 16)

        pltpu.sync_copy(bounds_hbm.at[wid], bndb)
        bv = bndb[...]
        c0 = jnp.sum(jnp.where(iota == 0, bv, 0))
        c1 = jnp.sum(jnp.where(iota == 1, bv, 0))

        if first:
            zeros16 = jnp.zeros((16,), jnp.float32)

            @pl.loop(0, R * F, step=16)
            def _(i):
                acc[pl.ds(i, 16)] = zeros16
        else:
            pltpu.sync_copy(xi_hbm.at[pl.ds(off, R * F)], acc)

            @pl.loop(0, R * F, step=16)
            def _(i):
                acc[pl.ds(i, 16)] = -acc[pl.ds(i, 16)]

        def chunk_body(c, carry):
            eoff = c * CH
            pltpu.sync_copy(cols_hbm.at[pl.ds(eoff, CH)], colb)
            pltpu.sync_copy(rows_hbm.at[pl.ds(eoff, CH)], rowb)
            pltpu.sync_copy(vals_hbm.at[pl.ds(eoff, CH)], valb)
            gat = pltpu.async_copy(xg_hbm.at[colb], xbuf, sem)
            for g in range(CH // 16):
                rv = rowb[pl.ds(g * 16, 16)]
                vv = valb[pl.ds(g * 16, 16)]
                inr = (rv >= base) & (rv < base + R)
                veffb[pl.ds(g * 16, 16)] = jnp.where(inr, vv, 0.0)
                lrowb[pl.ds(g * 16, 16)] = jnp.clip(rv - base, 0, R - 1)
            gat.wait()

            def edge_body(e, ecarry):
                esplat = jnp.zeros((16,), jnp.int32) + e
                lrs = plsc.load_gather(lrowb, [esplat])
                vs = plsc.load_gather(veffb, [esplat])
                rowbase = lrs * F
                for j in range(F // 16):
                    xv = xbuf[e, pl.ds(j * 16, 16)]
                    idx = rowbase + (iota + j * 16)
                    plsc.addupdate_scatter(acc, [idx], xv * vs)
                return ecarry

            lax.fori_loop(0, CH, edge_body, 0, unroll=False)
            return carry

        lax.fori_loop(c0, c1, chunk_body, 0, unroll=False)

        pltpu.sync_copy(acc, xo_hbm.at[pl.ds(off, R * F)])

    return step


_sc_step_first = _make_sc_step(True)
_sc_step_rec = _make_sc_step(False)


def _matmul_body(x_ref, m_ref, o_ref):
    k = pl.program_id(1)

    @pl.when(k == 0)
    def _():
        o_ref[...] = jnp.zeros_like(o_ref)

    o_ref[...] += jnp.dot(x_ref[0], m_ref[0],
                          preferred_element_type=jnp.float32)


def _final_matmul(xs, m):
    bn = 3456
    grid = (N_PAD // bn, KS)
    return pl.pallas_call(
        _matmul_body,
        grid=grid,
        in_specs=[
            pl.BlockSpec((1, bn, F), lambda n, k: (k, n, 0)),
            pl.BlockSpec((1, F, F), lambda n, k: (k, 0, 0)),
        ],
        out_specs=pl.BlockSpec((bn, F), lambda n, k: (n, 0)),
        out_shape=jax.ShapeDtypeStruct((N_PAD, F), jnp.float32),
    )(xs, m)


def kernel(tensor, edge_row, edge_col, edge_val, W):
    Bv, Tv, Nv, Cv = tensor.shape
    BT = Bv * Tv
    x = jnp.reshape(tensor, (BT, Nv, Cv))
    x = jnp.transpose(x, (1, 2, 0))
    x0 = jnp.reshape(x, (Nv, Cv * BT))
    x0 = jnp.pad(x0, ((0, N_PAD - Nv), (0, 0)))

    order = jnp.argsort(edge_row)
    rows_s = jnp.concatenate(
        [edge_row[order], jnp.full((E_PAD - E,), N - 1, jnp.int32)])
    cols_s = jnp.concatenate(
        [edge_col[order], jnp.zeros((E_PAD - E,), jnp.int32)])
    vals_s = jnp.concatenate(
        [edge_val[order], jnp.zeros((E_PAD - E,), jnp.float32)])
    vals2_s = vals_s * 2.0

    tbase = jnp.arange(NW, dtype=jnp.int32) * R
    starts = jnp.searchsorted(rows_s, tbase, side="left").astype(jnp.int32)
    ends = jnp.searchsorted(rows_s, tbase + R, side="left").astype(jnp.int32)
    c0 = starts // CH
    c1 = (ends + CH - 1) // CH
    bounds = jnp.zeros((NW, 16), jnp.int32)
    bounds = bounds.at[:, 0].set(c0).at[:, 1].set(c1)

    xs = [x0]
    x1 = _sc_step_first(x0, x0.reshape(-1), cols_s, rows_s, vals_s,
                        bounds).reshape(N_PAD, F)
    xs.append(x1)
    xp0, xp1 = x0, x1
    for _ in range(2, KS):
        x2 = _sc_step_rec(xp1, xp0.reshape(-1), cols_s, rows_s, vals2_s,
                          bounds).reshape(N_PAD, F)
        xs.append(x2)
        xp0, xp1 = xp1, x2

    xk = jnp.stack(xs, axis=0)

    w3 = W.reshape(COUT, CIN, KS)
    m = jnp.einsum('ock,ij->kcioj', w3, jnp.eye(BT, dtype=W.dtype))
    m = m.reshape(KS, Cv * BT, COUT * BT)

    out = _final_matmul(xk, m)[:Nv]
    out = out.reshape(Nv, COUT, BT)
    out = jnp.transpose(out, (2, 0, 1))
    return out.reshape(Bv, Tv, Nv, COUT)


# speedup vs baseline: 2.6175x; 1.2625x over previous
import dataclasses
import functools

import jax
import jax.numpy as jnp
from jax import lax
from jax.experimental import pallas as pl
from jax.experimental.pallas import tpu as pltpu
from jax.experimental.pallas import tpu_sc as plsc

N = 10242
E = 71694
KS = 20
CIN = 16
COUT = 16
F = 256
N_PAD = 10368
R = 324
NW = 32
CH = 64
E_PAD = ((E + CH - 1) // CH) * CH

_mesh = plsc.VectorSubcoreMesh(core_axis_name="c", subcore_axis_name="s")

_sc_params = pltpu.CompilerParams()
if "needs_layout_passes" in pltpu.CompilerParams.__dataclass_fields__:
    _sc_params = dataclasses.replace(_sc_params, needs_layout_passes=False)


def _make_sc_step(first):
    @functools.partial(
        pl.kernel,
        mesh=_mesh,
        out_type=jax.ShapeDtypeStruct((N_PAD * F,), jnp.float32),
        scratch_types=[
            pltpu.VMEM((R * F,), jnp.float32),
            pltpu.VMEM((2, CH, F), jnp.float32),
            pltpu.VMEM((2, 3 * CH), jnp.int32),
            pltpu.VMEM((2, CH), jnp.int32),
            pltpu.VMEM((2, CH), jnp.float32),
            pltpu.VMEM((16,), jnp.int32),
            pltpu.SemaphoreType.DMA,
            pltpu.SemaphoreType.DMA,
            pltpu.SemaphoreType.DMA,
            pltpu.SemaphoreType.DMA,
        ],
        compiler_params=_sc_params,
    )
    def step(xg_hbm, xi_hbm, pk_hbm, bounds_hbm,
             xo_hbm, acc, xbuf, pkb, lrowb, veffb, bndb,
             isem0, isem1, gsem0, gsem1):
        isem = (isem0, isem1)
        gsem = (gsem0, gsem1)
        cid = lax.axis_index("c")
        sid = lax.axis_index("s")
        wid = sid * 2 + cid
        base = wid * R
        off = base * F
        iota = lax.iota(jnp.int32, 16)

        pltpu.sync_copy(bounds_hbm.at[wid], bndb)
        bv = bndb[...]
        c0 = jnp.sum(jnp.where(iota == 0, bv, 0))
        c1 = jnp.sum(jnp.where(iota == 1, bv, 0))
        cb = c1 - c0

        def idx_start(i, b):
            return pltpu.async_copy(pk_hbm.at[c0 + i], pkb.at[b], isem[b])

        def idx_wait(b):
            pltpu.make_async_copy(pk_hbm.at[c0], pkb.at[b], isem[b]).wait()

        def gat_start(b):
            return pltpu.async_copy(
                xg_hbm.at[pkb.at[b, pl.ds(0, CH)]], xbuf.at[b], gsem[b])

        def gat_wait(b):
            pltpu.make_async_copy(
                xg_hbm.at[pkb.at[b, pl.ds(0, CH)]], xbuf.at[b],
                gsem[b]).wait()

        if first:
            zeros16 = jnp.zeros((16,), jnp.float32)

            @pl.loop(0, R * F, step=16)
            def _(i):
                acc[pl.ds(i, 16)] = zeros16
        else:
            pltpu.sync_copy(xi_hbm.at[pl.ds(off, R * F)], acc)

            @pl.loop(0, R * F, step=16)
            def _(i):
                acc[pl.ds(i, 16)] = -acc[pl.ds(i, 16)]

        @pl.when(cb >= 1)
        def _():
            idx_start(0, 0)

        @pl.when(cb >= 2)
        def _():
            idx_start(1, 1)

        @pl.when(cb >= 1)
        def _():
            idx_wait(0)
            gat_start(0)

        def chunk_body(i, carry):
            b = lax.rem(i, 2)

            def run(b, nb):
                gat_wait(b)
                for g in range(CH // 16):
                    rv = pkb[b, pl.ds(CH + g * 16, 16)]
                    vb = pkb[b, pl.ds(2 * CH + g * 16, 16)]
                    vv = plsc.bitcast(vb, jnp.float32)
                    inr = (rv >= base) & (rv < base + R)
                    veffb[b, pl.ds(g * 16, 16)] = jnp.where(inr, vv, 0.0)
                    lrowb[b, pl.ds(g * 16, 16)] = jnp.clip(rv - base, 0,
                                                           R - 1)

                @pl.when(i + 2 < cb)
                def _():
                    idx_start(i + 2, b)

                @pl.when(i + 1 < cb)
                def _():
                    idx_wait(nb)
                    gat_start(nb)

                def edge_body(e, ecarry):
                    esplat = jnp.zeros((16,), jnp.int32) + e
                    lrs = plsc.load_gather(lrowb.at[b], [esplat])
                    vs = plsc.load_gather(veffb.at[b], [esplat])
                    rowbase = lrs * F
                    for j in range(F // 16):
                        xv = xbuf[b, e, pl.ds(j * 16, 16)]
                        idx = rowbase + (iota + j * 16)
                        plsc.addupdate_scatter(acc, [idx], xv * vs)
                    return ecarry

                lax.fori_loop(0, CH, edge_body, 0, unroll=False)

            @pl.when(b == 0)
            def _():
                run(0, 1)

            @pl.when(b == 1)
            def _():
                run(1, 0)

            return carry

        lax.fori_loop(0, cb, chunk_body, 0, unroll=False)

        pltpu.sync_copy(acc, xo_hbm.at[pl.ds(off, R * F)])

    return step


_sc_step_first = _make_sc_step(True)
_sc_step_rec = _make_sc_step(False)


def _matmul_body(x_ref, m_ref, o_ref):
    k = pl.program_id(1)

    @pl.when(k == 0)
    def _():
        o_ref[...] = jnp.zeros_like(o_ref)

    o_ref[...] += jnp.dot(x_ref[0], m_ref[0],
                          preferred_element_type=jnp.float32)


def _final_matmul(xs, m):
    bn = 3456
    grid = (N_PAD // bn, KS)
    return pl.pallas_call(
        _matmul_body,
        grid=grid,
        in_specs=[
            pl.BlockSpec((1, bn, F), lambda n, k: (k, n, 0)),
            pl.BlockSpec((1, F, F), lambda n, k: (k, 0, 0)),
        ],
        out_specs=pl.BlockSpec((bn, F), lambda n, k: (n, 0)),
        out_shape=jax.ShapeDtypeStruct((N_PAD, F), jnp.float32),
    )(xs, m)


def kernel(tensor, edge_row, edge_col, edge_val, W):
    Bv, Tv, Nv, Cv = tensor.shape
    BT = Bv * Tv
    x = jnp.reshape(tensor, (BT, Nv, Cv))
    x = jnp.transpose(x, (1, 2, 0))
    x0 = jnp.reshape(x, (Nv, Cv * BT))
    x0 = jnp.pad(x0, ((0, N_PAD - Nv), (0, 0)))

    order = jnp.argsort(edge_row)
    rows_s = jnp.concatenate(
        [edge_row[order], jnp.full((E_PAD - E,), N - 1, jnp.int32)])
    cols_s = jnp.concatenate(
        [edge_col[order], jnp.zeros((E_PAD - E,), jnp.int32)])
    vals_s = jnp.concatenate(
        [edge_val[order], jnp.zeros((E_PAD - E,), jnp.float32)])
    vals2_s = vals_s * 2.0

    nchunks = E_PAD // CH
    def pack(vals):
        vbits = lax.bitcast_convert_type(vals, jnp.int32)
        pk = jnp.stack([cols_s.reshape(nchunks, CH),
                        rows_s.reshape(nchunks, CH),
                        vbits.reshape(nchunks, CH)], axis=1)
        return pk.reshape(nchunks, 3 * CH)

    pk1 = pack(vals_s)
    pk2 = pack(vals2_s)

    tbase = jnp.arange(NW, dtype=jnp.int32) * R
    starts = jnp.searchsorted(rows_s, tbase, side="left").astype(jnp.int32)
    ends = jnp.searchsorted(rows_s, tbase + R, side="left").astype(jnp.int32)
    c0 = starts // CH
    c1 = (ends + CH - 1) // CH
    bounds = jnp.zeros((NW, 16), jnp.int32)
    bounds = bounds.at[:, 0].set(c0).at[:, 1].set(c1)

    xs = [x0]
    x1 = _sc_step_first(x0, x0.reshape(-1), pk1, bounds).reshape(N_PAD, F)
    xs.append(x1)
    xp0, xp1 = x0, x1
    for _ in range(2, KS):
        x2 = _sc_step_rec(xp1, xp0.reshape(-1), pk2,
                          bounds).reshape(N_PAD, F)
        xs.append(x2)
        xp0, xp1 = xp1, x2

    xk = jnp.stack(xs, axis=0)

    w3 = W.reshape(COUT, CIN, KS)
    m = jnp.einsum('ock,ij->kcioj', w3, jnp.eye(BT, dtype=W.dtype))
    m = m.reshape(KS, Cv * BT, COUT * BT)

    out = _final_matmul(xk, m)[:Nv]
    out = out.reshape(Nv, COUT, BT)
    out = jnp.transpose(out, (2, 0, 1))
    return out.reshape(Bv, Tv, Nv, COUT)
